# baseline (speedup 1.0000x reference)
import functools

import jax
import jax.numpy as jnp
from jax import lax
from jax.experimental import pallas as pl
from jax.experimental.pallas import tpu as pltpu
from jax.experimental.pallas import tpu_sc as plsc

D = 64
B = 16384
V = 1000000
L = 16
NC = 2
NS = 16
NW = NC * NS
ROWS_PER_W = B // NW
NBLK = (V + 127) // 128
BLK_PER_W = NBLK // NW
BLK_EXTRA = NBLK - BLK_PER_W * NW
RING = 4
MAXWIN = (BLK_PER_W + 1 + RING - 1) // RING + 1
SROWS = B + 2 * L
EPS = 1e-6


def _sqrt16(x):
    i = plsc.bitcast(x, jnp.int32)
    y = plsc.bitcast((i >> 1) + jnp.int32(0x1FBD1DF5), jnp.float32)
    half = jnp.float32(0.5)
    y = half * (y + x / y)
    y = half * (y + x / y)
    y = half * (y + x / y)
    return y


def _gather_body(u_idx, i_idx, u_tabT, i_tabT, stg_u, stg_i,
                 all_idx, wl_k, wl_cb, ring, batch, kidx, sem):
    wid = lax.axis_index("s") * NC + lax.axis_index("c")
    extra = jnp.minimum(wid, BLK_EXTRA)
    lo = wid * BLK_PER_W + extra
    nblk = BLK_PER_W + jnp.where(wid < BLK_EXTRA, 1, 0)
    hi = lo + nblk
    lanes = lax.iota(jnp.int32, L)

    for tab_ref, stg_ref, idx_ref in (
        (u_tabT, stg_u, u_idx),
        (i_tabT, stg_i, i_idx),
    ):
        pltpu.sync_copy(idx_ref, all_idx)

        def scan(q, cur):
            v = all_idx[pl.ds(q * L, L)]
            blk = v >> 7
            m = (blk >= lo) & (blk < hi)
            n = plsc.all_reduce_population_count(m)[0]
            kv = q * L + lanes
            cb = (v & 127) | (blk << 8)
            plsc.store_compressed(wl_k.at[pl.ds(cur, L)], kv, mask=m)
            plsc.store_compressed(wl_cb.at[pl.ds(cur, L)], cb, mask=m)
            return cur + n

        nwl = lax.fori_loop(0, B // L, scan, jnp.int32(0))
        nwlv = (nwl + L - 1) // L

        def window(wi, _):
            wb = lo + wi * RING
            we = jnp.minimum(wb + RING, hi)

            @pl.when(wb < hi)
            def _():
                for r in range(RING):
                    @pl.when(wb + r < hi)
                    def _(r=r):
                        start = pl.multiple_of((wb + r) * 128, 128)
                        pltpu.async_copy(
                            tab_ref.at[:, pl.ds(start, 128)],
                            ring.at[r], sem)
                for r in range(RING):
                    @pl.when(wb + r < hi)
                    def _(r=r):
                        pltpu.make_async_copy(
                            tab_ref.at[:, pl.ds(0, 128)],
                            ring.at[r], sem).wait()

                def scan_hits(q, _):
                    kv = wl_k[pl.ds(q * L, L)]
                    cb = wl_cb[pl.ds(q * L, L)]
                    blk = cb >> 8
                    valid = (q * L + lanes) < nwl
                    m = (blk >= wb) & (blk < we) & valid
                    n = plsc.all_reduce_population_count(m)[0]

                    @pl.when(n > 0)
                    def _():
                        slot = jnp.where(m, blk - wb, 0)
                        col = jnp.where(m, cb & 255, 0)
                        ksel = jnp.where(m, kv, B + lanes)
                        for f in range(D):
                            fv = jnp.full((L,), f, jnp.int32)
                            vals = plsc.load_gather(ring, [slot, fv, col])
                            plsc.store_scatter(batch, [lanes, fv], vals)
                        kidx[...] = ksel
                        pltpu.async_copy(
                            batch, stg_ref.at[kidx], sem).wait()
                    return 0

                lax.fori_loop(0, nwlv, scan_hits, 0)
            return 0

        lax.fori_loop(0, MAXWIN, window, 0)


def _dist_body(stg_u, stg_i, out_hbm, buf_u, buf_i, out_v, sem_u, sem_i):
    wid = lax.axis_index("s") * NC + lax.axis_index("c")
    base = wid * ROWS_PER_W
    lanes = lax.iota(jnp.int32, L)
    CH = 256
    for ch in range(ROWS_PER_W // CH):
        cu = pltpu.async_copy(
            stg_u.at[pl.ds(base + ch * CH, CH)], buf_u, sem_u)
        ci = pltpu.async_copy(
            stg_i.at[pl.ds(base + ch * CH, CH)], buf_i, sem_i)
        cu.wait()
        ci.wait()

        def group(g, _, ch=ch):
            rv = g * L + lanes
            acc = jnp.zeros((L,), jnp.float32)
            for f in range(D):
                fv = jnp.full((L,), f, jnp.int32)
                u = plsc.load_gather(buf_u, [rv, fv])
                v = plsc.load_gather(buf_i, [rv, fv])
                d = (u - v) + jnp.float32(EPS)
                acc = acc + d * d
            out_v[pl.ds(ch * CH + g * L, L)] = _sqrt16(acc)
            return 0

        lax.fori_loop(0, CH // L, group, 0)

    pltpu.sync_copy(out_v, out_hbm.at[pl.ds(base, ROWS_PER_W)])


_MESH = plsc.VectorSubcoreMesh(core_axis_name="c", subcore_axis_name="s")
_PARAMS = pltpu.CompilerParams(needs_layout_passes=False)

_gather_phase = functools.partial(
    pl.kernel,
    mesh=_MESH,
    out_type=(
        jax.ShapeDtypeStruct((SROWS, 128), jnp.float32),
        jax.ShapeDtypeStruct((SROWS, 128), jnp.float32),
    ),
    compiler_params=_PARAMS,
    scratch_types=[
        pltpu.VMEM((B,), jnp.int32),
        pltpu.VMEM((B + L,), jnp.int32),
        pltpu.VMEM((B + L,), jnp.int32),
        pltpu.VMEM((RING, D, 128), jnp.float32),
        pltpu.VMEM((L, 128), jnp.float32),
        pltpu.VMEM((L,), jnp.int32),
        pltpu.SemaphoreType.DMA,
    ],
)(lambda u_idx, i_idx, u_tabT, i_tabT, stg_u, stg_i, *scratch:
  _gather_body(u_idx, i_idx, u_tabT, i_tabT, stg_u, stg_i, *scratch))

_dist_phase = functools.partial(
    pl.kernel,
    mesh=_MESH,
    out_type=jax.ShapeDtypeStruct((B,), jnp.float32),
    compiler_params=_PARAMS,
    scratch_types=[
        pltpu.VMEM((256, 128), jnp.float32),
        pltpu.VMEM((256, 128), jnp.float32),
        pltpu.VMEM((ROWS_PER_W,), jnp.float32),
        pltpu.SemaphoreType.DMA,
        pltpu.SemaphoreType.DMA,
    ],
)(lambda stg_u, stg_i, out, *scratch: _dist_body(stg_u, stg_i, out, *scratch))


def kernel(U, I, UEmb_weight, IEmb_weight):
    stg_u, stg_i = _gather_phase(U, I, UEmb_weight.T, IEmb_weight.T)
    return _dist_phase(stg_u, stg_i)
